# Initial kernel scaffold; baseline (speedup 1.0000x reference)
import jax
import jax.numpy as jnp
from jax.experimental import pallas as pl


def kernel(x_atom, x_bond, x_atom_index, x_bond_index, x_mask, x_chemical_info, has_gpu, params):
    raise NotImplementedError("write your pallas kernel here")


# speedup vs baseline: 10.9236x; 10.9236x over previous
import jax
import jax.numpy as jnp
from jax.experimental import pallas as pl
from jax.experimental.pallas import tpu as pltpu

_B, _L, _NB, _NBOND = 128, 128, 8, 256
_AF, _BF, _FP = 39, 10, 128
_AFP, _BFP = 48, 16
_RADIUS, _T = 3, 2
_CHEM_IN = 200
F32 = jnp.float32


def _lrelu(x):
    return jnp.where(x >= 0, x, 0.01 * x)


def _elu(x):
    return jnp.where(x > 0, x, jnp.exp(jnp.minimum(x, 0.0)) - 1.0)


def _gru(x, h, wih_t, whh_t, bih, bhh):
    gi = jnp.dot(x, wih_t[...], preferred_element_type=F32) + bih[...]
    gh = jnp.dot(h, whh_t[...], preferred_element_type=F32) + bhh[...]
    r = jax.nn.sigmoid(gi[:, :_FP] + gh[:, :_FP])
    z = jax.nn.sigmoid(gi[:, _FP:2 * _FP] + gh[:, _FP:2 * _FP])
    n = jnp.tanh(gi[:, 2 * _FP:] + r * gh[:, 2 * _FP:])
    return (1.0 - z) * n + z * h


def _mol_kernel(xa_ref, xb_ref, ai_ref, bi_ref, xm_ref, *rest):
    out_ref = rest[-1]
    w = rest[:-1]
    wafc, bafc, wna, wnb, bnf = w[0:5]
    rw = [w[5 + 9 * r: 5 + 9 * (r + 1)] for r in range(_RADIUS)]
    (mw1, mw2, mbal, wmat, bmat, wmih, wmhh, mbih, mbhh) = w[5 + 9 * _RADIUS:]

    xa = xa_ref[0]
    xb = xb_ref[0]
    ai = ai_ref[0]
    bi = bi_ref[0]
    xm = xm_ref[0]

    iota_l = jax.lax.broadcasted_iota(jnp.int32, (1, _L), 1)
    iota_b = jax.lax.broadcasted_iota(jnp.int32, (1, _NBOND), 1)
    oneh_a = jnp.concatenate(
        [(ai[:, k:k + 1] == iota_l).astype(F32) for k in range(_NB)], axis=0)
    oneh_b = jnp.concatenate(
        [(bi[:, k:k + 1] == iota_b).astype(F32) for k in range(_NB)], axis=0)

    amask = (ai != _L - 1).astype(F32)
    smask = jnp.where(ai == _L - 1, -9e8, 0.0).astype(F32)

    atom_feature = _lrelu(jnp.dot(xa, wafc[...], preferred_element_type=F32) + bafc[...])
    na_proj = jnp.dot(xa, wna[...], preferred_element_type=F32)
    nb_proj = jnp.dot(xb, wnb[...], preferred_element_type=F32)
    nf = _lrelu(jnp.dot(oneh_a, na_proj, preferred_element_type=F32)
                + jnp.dot(oneh_b, nb_proj, preferred_element_type=F32)
                + bnf[...])

    h = atom_feature
    act = None
    for r in range(_RADIUS):
        w1, w2, bal, wat, bat, wih, whh, bih, bhh = rw[r]
        if r == 0:
            g = nf
            self_feat = atom_feature
        else:
            g = jnp.dot(oneh_a, act, preferred_element_type=F32)
            self_feat = act
        nft = jnp.dot(g, wat[...], preferred_element_type=F32) + bat[...]
        s_self = jnp.dot(self_feat, w1[...], preferred_element_type=F32)
        s_nb_flat = jnp.dot(g, w2[...], preferred_element_type=F32)
        s_nb = jnp.concatenate(
            [s_nb_flat[k * _L:(k + 1) * _L] for k in range(_NB)], axis=1)
        score = _lrelu(s_self + s_nb + bal[...]) + smask
        m = jnp.max(score, axis=1, keepdims=True)
        e = jnp.exp(score - m)
        aw = e / jnp.sum(e, axis=1, keepdims=True) * amask
        ctx = aw[:, 0:1] * nft[0:_L]
        for k in range(1, _NB):
            ctx = ctx + aw[:, k:k + 1] * nft[k * _L:(k + 1) * _L]
        ctx = _elu(ctx)
        h = _gru(ctx, h, wih, whh, bih, bhh)
        act = jnp.maximum(h, 0.0)

    mf = jnp.sum(act * xm, axis=0, keepdims=True)
    act_mol = jnp.maximum(mf, 0.0)
    msmask = jnp.where(xm == 0.0, -9e8, 0.0)
    for _t in range(_T):
        s1 = jnp.dot(act_mol, mw1[...], preferred_element_type=F32)
        s2 = jnp.dot(act, mw2[...], preferred_element_type=F32)
        ms = _lrelu(s1 + s2 + mbal[...]) + msmask
        m = jnp.max(ms, axis=0, keepdims=True)
        e = jnp.exp(ms - m)
        maw = e / jnp.sum(e, axis=0, keepdims=True) * xm
        aft = jnp.dot(act, wmat[...], preferred_element_type=F32) + bmat[...]
        mctx = _elu(jnp.sum(maw * aft, axis=0, keepdims=True))
        mf = _gru(mctx, mf, wmih, wmhh, mbih, mbhh)
        act_mol = jnp.maximum(mf, 0.0)
    out_ref[0] = mf


def _bn(x, g, b, eps=1e-5):
    m = jnp.mean(x, axis=0, keepdims=True)
    v = jnp.mean((x - m) * (x - m), axis=0, keepdims=True)
    return (x - m) / jnp.sqrt(v + eps) * g + b


def _head_kernel(mf_ref, chem_ref, gmn, bmn, wmo, bmo,
                 f1w, f1b, g1, b1, f2w, f2b, g2, b2, f3w, f3b, g3, b3,
                 f4w, f4b, g4, b4, f5w, f5b, owt, owb, ob, out_ref):
    mf = mf_ref[...]
    chem = chem_ref[...]
    mol_pred = jnp.dot(_bn(mf, gmn[...], bmn[...]), wmo[...],
                       preferred_element_type=F32) + bmo[...]
    y = jnp.maximum(_bn(jnp.dot(chem, f1w[...], preferred_element_type=F32) + f1b[...],
                        g1[...], b1[...]), 0.0)
    y = jnp.maximum(_bn(jnp.dot(y, f2w[...], preferred_element_type=F32) + f2b[...],
                        g2[...], b2[...]), 0.0)
    y = jnp.maximum(_bn(jnp.dot(y, f3w[...], preferred_element_type=F32) + f3b[...],
                        g3[...], b3[...]), 0.0)
    y = jnp.maximum(_bn(jnp.dot(y, f4w[...], preferred_element_type=F32) + f4b[...],
                        g4[...], b4[...]), 0.0)
    y = jnp.dot(y, f5w[...], preferred_element_type=F32) + f5b[...]
    o = (jnp.dot(mol_pred, owt[...], preferred_element_type=F32)
         + jnp.dot(y, owb[...], preferred_element_type=F32) + ob[...])
    out_ref[...] = jax.nn.sigmoid(o)


def kernel(x_atom, x_bond, x_atom_index, x_bond_index, x_mask, x_chemical_info, has_gpu, params):
    p = params
    xa = jnp.pad(x_atom, ((0, 0), (0, 0), (0, _AFP - _AF)))
    xb = jnp.pad(x_bond, ((0, 0), (0, 0), (0, _BFP - _BF)))
    xm3 = x_mask.reshape(_B, _L, 1)

    def row(v):
        return v.reshape(1, -1)

    weights = [
        jnp.pad(p['atom_fc_w'], ((0, _AFP - _AF), (0, 0))), row(p['atom_fc_b']),
        jnp.pad(p['neighbor_fc_w'][:_AF], ((0, _AFP - _AF), (0, 0))),
        jnp.pad(p['neighbor_fc_w'][_AF:], ((0, _BFP - _BF), (0, 0))),
        row(p['neighbor_fc_b']),
    ]
    for r in range(_RADIUS):
        weights += [
            p['align_w'][r][:_FP], p['align_w'][r][_FP:], row(p['align_b'][r]),
            p['attend_w'][r], row(p['attend_b'][r]),
            p['gru_wih'][r].T, p['gru_whh'][r].T,
            row(p['gru_bih'][r]), row(p['gru_bhh'][r]),
        ]
    weights += [
        p['mol_align_w'][:_FP], p['mol_align_w'][_FP:], row(p['mol_align_b']),
        p['mol_attend_w'], row(p['mol_attend_b']),
        p['mol_gru_wih'].T, p['mol_gru_whh'].T,
        row(p['mol_gru_bih']), row(p['mol_gru_bhh']),
    ]

    data_specs = [
        pl.BlockSpec((1, _L, _AFP), lambda b: (b, 0, 0)),
        pl.BlockSpec((1, _NBOND, _BFP), lambda b: (b, 0, 0)),
        pl.BlockSpec((1, _L, _NB), lambda b: (b, 0, 0)),
        pl.BlockSpec((1, _L, _NB), lambda b: (b, 0, 0)),
        pl.BlockSpec((1, _L, 1), lambda b: (b, 0, 0)),
    ]
    wspecs = [pl.BlockSpec(wt.shape, lambda b, _n=wt.ndim: (0,) * _n)
              for wt in weights]

    mf = pl.pallas_call(
        _mol_kernel,
        grid=(_B,),
        in_specs=data_specs + wspecs,
        out_specs=pl.BlockSpec((1, 1, _FP), lambda b: (b, 0, 0)),
        out_shape=jax.ShapeDtypeStruct((_B, 1, _FP), F32),
        compiler_params=pltpu.CompilerParams(
            dimension_semantics=("arbitrary",)),
    )(xa, xb, x_atom_index, x_bond_index, xm3, *weights)
    mf = mf.reshape(_B, _FP)

    head_inputs = [
        mf, x_chemical_info,
        row(p['mol_norm_g']), row(p['mol_norm_b']),
        p['mol_output_w'], row(p['mol_output_b']),
        p['fc1_w'], row(p['fc1_b']), row(p['bn1_g']), row(p['bn1_b']),
        p['fc2_w'], row(p['fc2_b']), row(p['bn2_g']), row(p['bn2_b']),
        p['fc3_w'], row(p['fc3_b']), row(p['bn3_g']), row(p['bn3_b']),
        p['fc4_w'], row(p['fc4_b']), row(p['bn4_g']), row(p['bn4_b']),
        p['fc5_w'], row(p['fc5_b']),
        p['out_w'][:_FP], p['out_w'][_FP:], row(p['out_b']),
    ]
    out = pl.pallas_call(
        _head_kernel,
        out_shape=jax.ShapeDtypeStruct((_B, 1), F32),
    )(*head_inputs)
    return out


# speedup vs baseline: 11.1606x; 1.0217x over previous
import jax
import jax.numpy as jnp
from jax.experimental import pallas as pl
from jax.experimental.pallas import tpu as pltpu

_B, _L, _NB, _NBOND = 128, 128, 8, 256
_AF, _BF, _FP = 39, 10, 128
_AFP, _BFP = 48, 16
_RADIUS, _T = 3, 2
_CHEM_IN = 200
F32 = jnp.float32


def _lrelu(x):
    return jnp.where(x >= 0, x, 0.01 * x)


def _elu(x):
    return jnp.where(x > 0, x, jnp.exp(jnp.minimum(x, 0.0)) - 1.0)


def _gru(x, h, wih_t, whh_t, bih, bhh):
    gi = jnp.dot(x, wih_t[...], preferred_element_type=F32) + bih[...]
    gh = jnp.dot(h, whh_t[...], preferred_element_type=F32) + bhh[...]
    r = jax.nn.sigmoid(gi[:, :_FP] + gh[:, :_FP])
    z = jax.nn.sigmoid(gi[:, _FP:2 * _FP] + gh[:, _FP:2 * _FP])
    n = jnp.tanh(gi[:, 2 * _FP:] + r * gh[:, 2 * _FP:])
    return (1.0 - z) * n + z * h


def _mol_kernel(xa_ref, xb_ref, ai_ref, bi_ref, xm_ref, *rest):
    out_ref = rest[-1]
    w = rest[:-1]
    wafc, bafc, wna, wnb, bnf = w[0:5]
    rw = [w[5 + 9 * r: 5 + 9 * (r + 1)] for r in range(_RADIUS)]
    (mw1, mw2, mbal, wmat, bmat, wmih, wmhh, mbih, mbhh) = w[5 + 9 * _RADIUS:]

    xa = xa_ref[0]
    xb = xb_ref[0]
    ai = ai_ref[0]
    bi = bi_ref[0]
    xm = xm_ref[0]

    iota_l = jax.lax.broadcasted_iota(jnp.int32, (1, _L), 1)
    iota_b = jax.lax.broadcasted_iota(jnp.int32, (1, _NBOND), 1)
    oneh_a = jnp.concatenate(
        [(ai[:, k:k + 1] == iota_l).astype(F32) for k in range(_NB)], axis=0)
    oneh_b = jnp.concatenate(
        [(bi[:, k:k + 1] == iota_b).astype(F32) for k in range(_NB)], axis=0)

    amask = (ai != _L - 1).astype(F32)
    smask = jnp.where(ai == _L - 1, -9e8, 0.0).astype(F32)

    atom_feature = _lrelu(jnp.dot(xa, wafc[...], preferred_element_type=F32) + bafc[...])
    na_proj = jnp.dot(xa, wna[...], preferred_element_type=F32)
    nb_proj = jnp.dot(xb, wnb[...], preferred_element_type=F32)
    nf = _lrelu(jnp.dot(oneh_a, na_proj, preferred_element_type=F32)
                + jnp.dot(oneh_b, nb_proj, preferred_element_type=F32)
                + bnf[...])

    h = atom_feature
    act = None
    for r in range(_RADIUS):
        w1, w2, bal, wat, bat, wih, whh, bih, bhh = rw[r]
        if r == 0:
            nft = jnp.dot(nf, wat[...], preferred_element_type=F32) + bat[...]
            s_self = jnp.dot(atom_feature, w1[...], preferred_element_type=F32)
            s_nb_flat = jnp.dot(nf, w2[...], preferred_element_type=F32)
        else:
            at_proj = jnp.dot(act, wat[...], preferred_element_type=F32)
            s2_proj = jnp.dot(act, w2[...], preferred_element_type=F32)
            nft = jnp.dot(oneh_a, at_proj, preferred_element_type=F32) + bat[...]
            s_self = jnp.dot(act, w1[...], preferred_element_type=F32)
            s_nb_flat = jnp.dot(oneh_a, s2_proj, preferred_element_type=F32)
        s_nb = jnp.concatenate(
            [s_nb_flat[k * _L:(k + 1) * _L] for k in range(_NB)], axis=1)
        score = _lrelu(s_self + s_nb + bal[...]) + smask
        m = jnp.max(score, axis=1, keepdims=True)
        e = jnp.exp(score - m)
        aw = e / jnp.sum(e, axis=1, keepdims=True) * amask
        ctx = aw[:, 0:1] * nft[0:_L]
        for k in range(1, _NB):
            ctx = ctx + aw[:, k:k + 1] * nft[k * _L:(k + 1) * _L]
        ctx = _elu(ctx)
        h = _gru(ctx, h, wih, whh, bih, bhh)
        act = jnp.maximum(h, 0.0)

    mf = jnp.sum(act * xm, axis=0, keepdims=True)
    act_mol = jnp.maximum(mf, 0.0)
    msmask = jnp.where(xm == 0.0, -9e8, 0.0)
    for _t in range(_T):
        s1 = jnp.dot(act_mol, mw1[...], preferred_element_type=F32)
        s2 = jnp.dot(act, mw2[...], preferred_element_type=F32)
        ms = _lrelu(s1 + s2 + mbal[...]) + msmask
        m = jnp.max(ms, axis=0, keepdims=True)
        e = jnp.exp(ms - m)
        maw = e / jnp.sum(e, axis=0, keepdims=True) * xm
        aft = jnp.dot(act, wmat[...], preferred_element_type=F32) + bmat[...]
        mctx = _elu(jnp.sum(maw * aft, axis=0, keepdims=True))
        mf = _gru(mctx, mf, wmih, wmhh, mbih, mbhh)
        act_mol = jnp.maximum(mf, 0.0)
    out_ref[0] = mf


def _bn(x, g, b, eps=1e-5):
    m = jnp.mean(x, axis=0, keepdims=True)
    v = jnp.mean((x - m) * (x - m), axis=0, keepdims=True)
    return (x - m) / jnp.sqrt(v + eps) * g + b


def _head_kernel(mf_ref, chem_ref, gmn, bmn, wmo, bmo,
                 f1w, f1b, g1, b1, f2w, f2b, g2, b2, f3w, f3b, g3, b3,
                 f4w, f4b, g4, b4, f5w, f5b, owt, owb, ob, out_ref):
    mf = mf_ref[...]
    chem = chem_ref[...]
    mol_pred = jnp.dot(_bn(mf, gmn[...], bmn[...]), wmo[...],
                       preferred_element_type=F32) + bmo[...]
    y = jnp.maximum(_bn(jnp.dot(chem, f1w[...], preferred_element_type=F32) + f1b[...],
                        g1[...], b1[...]), 0.0)
    y = jnp.maximum(_bn(jnp.dot(y, f2w[...], preferred_element_type=F32) + f2b[...],
                        g2[...], b2[...]), 0.0)
    y = jnp.maximum(_bn(jnp.dot(y, f3w[...], preferred_element_type=F32) + f3b[...],
                        g3[...], b3[...]), 0.0)
    y = jnp.maximum(_bn(jnp.dot(y, f4w[...], preferred_element_type=F32) + f4b[...],
                        g4[...], b4[...]), 0.0)
    y = jnp.dot(y, f5w[...], preferred_element_type=F32) + f5b[...]
    o = (jnp.dot(mol_pred, owt[...], preferred_element_type=F32)
         + jnp.dot(y, owb[...], preferred_element_type=F32) + ob[...])
    out_ref[...] = jax.nn.sigmoid(o)


def kernel(x_atom, x_bond, x_atom_index, x_bond_index, x_mask, x_chemical_info, has_gpu, params):
    p = params
    xa = jnp.pad(x_atom, ((0, 0), (0, 0), (0, _AFP - _AF)))
    xb = jnp.pad(x_bond, ((0, 0), (0, 0), (0, _BFP - _BF)))
    xm3 = x_mask.reshape(_B, _L, 1)

    def row(v):
        return v.reshape(1, -1)

    weights = [
        jnp.pad(p['atom_fc_w'], ((0, _AFP - _AF), (0, 0))), row(p['atom_fc_b']),
        jnp.pad(p['neighbor_fc_w'][:_AF], ((0, _AFP - _AF), (0, 0))),
        jnp.pad(p['neighbor_fc_w'][_AF:], ((0, _BFP - _BF), (0, 0))),
        row(p['neighbor_fc_b']),
    ]
    for r in range(_RADIUS):
        weights += [
            p['align_w'][r][:_FP], p['align_w'][r][_FP:], row(p['align_b'][r]),
            p['attend_w'][r], row(p['attend_b'][r]),
            p['gru_wih'][r].T, p['gru_whh'][r].T,
            row(p['gru_bih'][r]), row(p['gru_bhh'][r]),
        ]
    weights += [
        p['mol_align_w'][:_FP], p['mol_align_w'][_FP:], row(p['mol_align_b']),
        p['mol_attend_w'], row(p['mol_attend_b']),
        p['mol_gru_wih'].T, p['mol_gru_whh'].T,
        row(p['mol_gru_bih']), row(p['mol_gru_bhh']),
    ]

    data_specs = [
        pl.BlockSpec((1, _L, _AFP), lambda b: (b, 0, 0)),
        pl.BlockSpec((1, _NBOND, _BFP), lambda b: (b, 0, 0)),
        pl.BlockSpec((1, _L, _NB), lambda b: (b, 0, 0)),
        pl.BlockSpec((1, _L, _NB), lambda b: (b, 0, 0)),
        pl.BlockSpec((1, _L, 1), lambda b: (b, 0, 0)),
    ]
    wspecs = [pl.BlockSpec(wt.shape, lambda b, _n=wt.ndim: (0,) * _n)
              for wt in weights]

    mf = pl.pallas_call(
        _mol_kernel,
        grid=(_B,),
        in_specs=data_specs + wspecs,
        out_specs=pl.BlockSpec((1, 1, _FP), lambda b: (b, 0, 0)),
        out_shape=jax.ShapeDtypeStruct((_B, 1, _FP), F32),
        compiler_params=pltpu.CompilerParams(
            dimension_semantics=("arbitrary",)),
    )(xa, xb, x_atom_index, x_bond_index, xm3, *weights)
    mf = mf.reshape(_B, _FP)

    head_inputs = [
        mf, x_chemical_info,
        row(p['mol_norm_g']), row(p['mol_norm_b']),
        p['mol_output_w'], row(p['mol_output_b']),
        p['fc1_w'], row(p['fc1_b']), row(p['bn1_g']), row(p['bn1_b']),
        p['fc2_w'], row(p['fc2_b']), row(p['bn2_g']), row(p['bn2_b']),
        p['fc3_w'], row(p['fc3_b']), row(p['bn3_g']), row(p['bn3_b']),
        p['fc4_w'], row(p['fc4_b']), row(p['bn4_g']), row(p['bn4_b']),
        p['fc5_w'], row(p['fc5_b']),
        p['out_w'][:_FP], p['out_w'][_FP:], row(p['out_b']),
    ]
    out = pl.pallas_call(
        _head_kernel,
        out_shape=jax.ShapeDtypeStruct((_B, 1), F32),
    )(*head_inputs)
    return out


# speedup vs baseline: 11.3269x; 1.0149x over previous
import jax
import jax.numpy as jnp
from jax.experimental import pallas as pl
from jax.experimental.pallas import tpu as pltpu

_B, _L, _NB, _NBOND = 128, 128, 8, 256
_AF, _BF, _FP = 39, 10, 128
_AFP, _BFP = 48, 16
_RADIUS, _T = 3, 2
_CHEM_IN = 200
_BM = 2
F32 = jnp.float32


def _lrelu(x):
    return jnp.where(x >= 0, x, 0.01 * x)


def _elu(x):
    return jnp.where(x > 0, x, jnp.exp(jnp.minimum(x, 0.0)) - 1.0)


def _gru(x, h, wih_t, whh_t, bih, bhh):
    gi = jnp.dot(x, wih_t[...], preferred_element_type=F32) + bih[...]
    gh = jnp.dot(h, whh_t[...], preferred_element_type=F32) + bhh[...]
    r = jax.nn.sigmoid(gi[:, :_FP] + gh[:, :_FP])
    z = jax.nn.sigmoid(gi[:, _FP:2 * _FP] + gh[:, _FP:2 * _FP])
    n = jnp.tanh(gi[:, 2 * _FP:] + r * gh[:, 2 * _FP:])
    return (1.0 - z) * n + z * h


_NW_G, _NW_R0, _NW_R, _NW_M = 5, 8, 7, 8


def _one_mol(xa, xb, ai, bi, xm, w):
    wafc, bafc, wna, wnb, bnf = w[:_NW_G]
    w0 = w[_NW_G:_NW_G + _NW_R0]
    wr = [w[_NW_G + _NW_R0 + _NW_R * i: _NW_G + _NW_R0 + _NW_R * (i + 1)]
          for i in range(_RADIUS - 1)]
    mcat, mw1, mbal, bmat, wmih, wmhh, mbih, mbhh = w[-_NW_M:]

    iota_l = jax.lax.broadcasted_iota(jnp.int32, (1, _L), 1)
    iota_b = jax.lax.broadcasted_iota(jnp.int32, (1, _NBOND), 1)
    oneh_a = jnp.concatenate(
        [(ai[:, k:k + 1] == iota_l).astype(F32) for k in range(_NB)], axis=0)
    oneh_b = jnp.concatenate(
        [(bi[:, k:k + 1] == iota_b).astype(F32) for k in range(_NB)], axis=0)

    amask = (ai != _L - 1).astype(F32)
    smask = jnp.where(ai == _L - 1, -9e8, 0.0).astype(F32)

    atom_feature = _lrelu(jnp.dot(xa, wafc[...], preferred_element_type=F32) + bafc[...])
    na_proj = jnp.dot(xa, wna[...], preferred_element_type=F32)
    nb_proj = jnp.dot(xb, wnb[...], preferred_element_type=F32)
    nf = _lrelu(jnp.dot(oneh_a, na_proj, preferred_element_type=F32)
                + jnp.dot(oneh_b, nb_proj, preferred_element_type=F32)
                + bnf[...])

    h = atom_feature
    act = None
    for r in range(_RADIUS):
        if r == 0:
            wcat0, w1, bal, bat, wih, whh, bih, bhh = w0
            cat = jnp.dot(nf, wcat0[...], preferred_element_type=F32)
            nft = cat[:, :_FP] + bat[...]
            s_nb_flat = cat[:, _FP:_FP + 1]
            s_self = jnp.dot(atom_feature, w1[...], preferred_element_type=F32)
        else:
            pcat, bal, bat, wih, whh, bih, bhh = wr[r - 1]
            proj = jnp.dot(act, pcat[...], preferred_element_type=F32)
            gath = jnp.dot(oneh_a, proj[:, :2 * _FP], preferred_element_type=F32)
            nft = gath[:, :_FP] + bat[...]
            s_nb_flat = gath[:, _FP:_FP + 1]
            s_self = proj[:, _FP + 1:_FP + 2]
        s_nb = jnp.concatenate(
            [s_nb_flat[k * _L:(k + 1) * _L] for k in range(_NB)], axis=1)
        score = _lrelu(s_self + s_nb + bal[...]) + smask
        m = jnp.max(score, axis=1, keepdims=True)
        e = jnp.exp(score - m)
        aw = e / jnp.sum(e, axis=1, keepdims=True) * amask
        ctx = aw[:, 0:1] * nft[0:_L]
        for k in range(1, _NB):
            ctx = ctx + aw[:, k:k + 1] * nft[k * _L:(k + 1) * _L]
        ctx = _elu(ctx)
        h = _gru(ctx, h, wih, whh, bih, bhh)
        act = jnp.maximum(h, 0.0)

    mf = jnp.sum(act * xm, axis=0, keepdims=True)
    act_mol = jnp.maximum(mf, 0.0)
    msmask = jnp.where(xm == 0.0, -9e8, 0.0)
    for _t in range(_T):
        s1 = jnp.dot(act_mol, mw1[...], preferred_element_type=F32)
        cam = jnp.dot(act, mcat[...], preferred_element_type=F32)
        aft = cam[:, :_FP] + bmat[...]
        s2 = cam[:, _FP:_FP + 1]
        ms = _lrelu(s1 + s2 + mbal[...]) + msmask
        m = jnp.max(ms, axis=0, keepdims=True)
        e = jnp.exp(ms - m)
        maw = e / jnp.sum(e, axis=0, keepdims=True) * xm
        mctx = _elu(jnp.sum(maw * aft, axis=0, keepdims=True))
        mf = _gru(mctx, mf, wmih, wmhh, mbih, mbhh)
        act_mol = jnp.maximum(mf, 0.0)
    return mf


def _mol_kernel(xa_ref, xb_ref, ai_ref, bi_ref, xm_ref, *rest):
    out_ref = rest[-1]
    w = rest[:-1]
    for mslot in range(_BM):
        mf = _one_mol(xa_ref[mslot], xb_ref[mslot], ai_ref[mslot],
                      bi_ref[mslot], xm_ref[mslot], w)
        out_ref[mslot] = mf


def _bn(x, g, b, eps=1e-5):
    m = jnp.mean(x, axis=0, keepdims=True)
    v = jnp.mean((x - m) * (x - m), axis=0, keepdims=True)
    return (x - m) / jnp.sqrt(v + eps) * g + b


def _head_kernel(mf_ref, chem_ref, gmn, bmn, wmo, bmo,
                 f1w, f1b, g1, b1, f2w, f2b, g2, b2, f3w, f3b, g3, b3,
                 f4w, f4b, g4, b4, f5w, f5b, owt, owb, ob, out_ref):
    mf = mf_ref[...]
    chem = chem_ref[...]
    mol_pred = jnp.dot(_bn(mf, gmn[...], bmn[...]), wmo[...],
                       preferred_element_type=F32) + bmo[...]
    y = jnp.maximum(_bn(jnp.dot(chem, f1w[...], preferred_element_type=F32) + f1b[...],
                        g1[...], b1[...]), 0.0)
    y = jnp.maximum(_bn(jnp.dot(y, f2w[...], preferred_element_type=F32) + f2b[...],
                        g2[...], b2[...]), 0.0)
    y = jnp.maximum(_bn(jnp.dot(y, f3w[...], preferred_element_type=F32) + f3b[...],
                        g3[...], b3[...]), 0.0)
    y = jnp.maximum(_bn(jnp.dot(y, f4w[...], preferred_element_type=F32) + f4b[...],
                        g4[...], b4[...]), 0.0)
    y = jnp.dot(y, f5w[...], preferred_element_type=F32) + f5b[...]
    o = (jnp.dot(mol_pred, owt[...], preferred_element_type=F32)
         + jnp.dot(y, owb[...], preferred_element_type=F32) + ob[...])
    out_ref[...] = jax.nn.sigmoid(o)


def kernel(x_atom, x_bond, x_atom_index, x_bond_index, x_mask, x_chemical_info, has_gpu, params):
    p = params
    xa = jnp.pad(x_atom, ((0, 0), (0, 0), (0, _AFP - _AF)))
    xb = jnp.pad(x_bond, ((0, 0), (0, 0), (0, _BFP - _BF)))
    xm3 = x_mask.reshape(_B, _L, 1)

    def row(v):
        return v.reshape(1, -1)

    weights = [
        jnp.pad(p['atom_fc_w'], ((0, _AFP - _AF), (0, 0))), row(p['atom_fc_b']),
        jnp.pad(p['neighbor_fc_w'][:_AF], ((0, _AFP - _AF), (0, 0))),
        jnp.pad(p['neighbor_fc_w'][_AF:], ((0, _BFP - _BF), (0, 0))),
        row(p['neighbor_fc_b']),
    ]
    wcat0 = jnp.pad(
        jnp.concatenate([p['attend_w'][0], p['align_w'][0][_FP:]], axis=1),
        ((0, 0), (0, _FP - 1)))
    weights += [
        wcat0, p['align_w'][0][:_FP], row(p['align_b'][0]), row(p['attend_b'][0]),
        p['gru_wih'][0].T, p['gru_whh'][0].T,
        row(p['gru_bih'][0]), row(p['gru_bhh'][0]),
    ]
    for r in range(1, _RADIUS):
        pcat = jnp.pad(
            jnp.concatenate([p['attend_w'][r], p['align_w'][r][_FP:],
                             p['align_w'][r][:_FP]], axis=1),
            ((0, 0), (0, _FP - 2)))
        weights += [
            pcat, row(p['align_b'][r]), row(p['attend_b'][r]),
            p['gru_wih'][r].T, p['gru_whh'][r].T,
            row(p['gru_bih'][r]), row(p['gru_bhh'][r]),
        ]
    mcat = jnp.pad(
        jnp.concatenate([p['mol_attend_w'], p['mol_align_w'][_FP:]], axis=1),
        ((0, 0), (0, _FP - 1)))
    weights += [
        mcat, p['mol_align_w'][:_FP], row(p['mol_align_b']), row(p['mol_attend_b']),
        p['mol_gru_wih'].T, p['mol_gru_whh'].T,
        row(p['mol_gru_bih']), row(p['mol_gru_bhh']),
    ]

    data_specs = [
        pl.BlockSpec((_BM, _L, _AFP), lambda b: (b, 0, 0)),
        pl.BlockSpec((_BM, _NBOND, _BFP), lambda b: (b, 0, 0)),
        pl.BlockSpec((_BM, _L, _NB), lambda b: (b, 0, 0)),
        pl.BlockSpec((_BM, _L, _NB), lambda b: (b, 0, 0)),
        pl.BlockSpec((_BM, _L, 1), lambda b: (b, 0, 0)),
    ]
    wspecs = [pl.BlockSpec(wt.shape, lambda b, _n=wt.ndim: (0,) * _n)
              for wt in weights]

    mf = pl.pallas_call(
        _mol_kernel,
        grid=(_B // _BM,),
        in_specs=data_specs + wspecs,
        out_specs=pl.BlockSpec((_BM, 1, _FP), lambda b: (b, 0, 0)),
        out_shape=jax.ShapeDtypeStruct((_B, 1, _FP), F32),
        compiler_params=pltpu.CompilerParams(
            dimension_semantics=("arbitrary",)),
    )(xa, xb, x_atom_index, x_bond_index, xm3, *weights)
    mf = mf.reshape(_B, _FP)

    head_inputs = [
        mf, x_chemical_info,
        row(p['mol_norm_g']), row(p['mol_norm_b']),
        p['mol_output_w'], row(p['mol_output_b']),
        p['fc1_w'], row(p['fc1_b']), row(p['bn1_g']), row(p['bn1_b']),
        p['fc2_w'], row(p['fc2_b']), row(p['bn2_g']), row(p['bn2_b']),
        p['fc3_w'], row(p['fc3_b']), row(p['bn3_g']), row(p['bn3_b']),
        p['fc4_w'], row(p['fc4_b']), row(p['bn4_g']), row(p['bn4_b']),
        p['fc5_w'], row(p['fc5_b']),
        p['out_w'][:_FP], p['out_w'][_FP:], row(p['out_b']),
    ]
    out = pl.pallas_call(
        _head_kernel,
        out_shape=jax.ShapeDtypeStruct((_B, 1), F32),
    )(*head_inputs)
    return out
